# baseline (speedup 1.0000x reference)
import functools

import jax
import jax.numpy as jnp
from jax import lax
from jax.experimental import pallas as pl
from jax.experimental.pallas import tpu as pltpu
from jax.experimental.pallas import tpu_sc as plsc

R = 32
R3 = R * R * R
B, C, N = 8, 32, 65536
HALF = N // 2
NW = 32
WPB = NW // B
CPW = C // WPB
CH = 4096
NCHUNK = HALF // CH
LANES = 16


def _normalize_body(coords_ref, norm_ref, w_ref):
    c = coords_ref[...]
    mean = jnp.mean(c, axis=2, keepdims=True)
    nc = c - mean
    r = jnp.max(jnp.sqrt(jnp.sum(nc * nc, axis=1, keepdims=True)),
                axis=2, keepdims=True)
    nc = nc / (r * 2.0) + 0.5
    nc = jnp.clip(nc * R, 0, R - 1)
    norm_ref[...] = nc
    vox = jnp.round(nc).astype(jnp.int32)
    idx = vox[:, 0, :] * (R * R) + vox[:, 1, :] * R + vox[:, 2, :]
    lo = idx[:, :HALF]
    hi = idx[:, HALF:]
    w_ref[...] = (lo | (hi << 16))[:, None, :]


_normalize = pl.pallas_call(
    _normalize_body,
    grid=(B,),
    in_specs=[pl.BlockSpec((1, 3, N), lambda b: (b, 0, 0))],
    out_specs=(
        pl.BlockSpec((1, 3, N), lambda b: (b, 0, 0)),
        pl.BlockSpec((1, 1, HALF), lambda b: (b, 0, 0)),
    ),
    out_shape=(
        jax.ShapeDtypeStruct((B, 3, N), jnp.float32),
        jax.ShapeDtypeStruct((B, 1, HALF), jnp.int32),
    ),
)


UNROLL = 16


def _scatter_body(feat_hbm, w_hbm, out_hbm, w_v, acc_v, recip_v, fa_v, fb_v,
                  sem_a, sem_b):
    wid = lax.axis_index("s") * 2 + lax.axis_index("c")
    b = wid // WPB
    j = wid % WPB

    pltpu.sync_copy(w_hbm.at[b], w_v)

    zeros = jnp.zeros((LANES,), jnp.float32)
    ones = jnp.ones((LANES,), jnp.float32)

    def zero_acc():
        @plsc.parallel_loop(0, R3, LANES, unroll=UNROLL)
        def _(i):
            acc_v[pl.ds(i, LANES)] = zeros

    zero_acc()

    @plsc.parallel_loop(0, HALF, LANES, unroll=UNROLL)
    def _(i):
        w = w_v[pl.ds(i, LANES)]
        ia = w & 0xFFFF
        ib = lax.shift_right_logical(w, 16)
        plsc.addupdate_scatter(acc_v, [ia], ones)
        plsc.addupdate_scatter(acc_v, [ib], ones)

    @plsc.parallel_loop(0, R3, LANES, unroll=UNROLL)
    def _(i):
        s = pl.ds(i, LANES)
        recip_v[s] = 1.0 / jnp.maximum(acc_v[s], 1.0)

    def chan_body(c8, carry):
        ch = j * CPW + c8
        zero_acc()

        def start_chunk(k):
            p = k % 2
            ca = pltpu.async_copy(feat_hbm.at[b, ch, pl.ds(k * CH, CH)],
                                  fa_v.at[p], sem_a if p == 0 else sem_b)
            cb = pltpu.async_copy(feat_hbm.at[b, ch, pl.ds(HALF + k * CH, CH)],
                                  fb_v.at[p], sem_a if p == 0 else sem_b)
            return ca, cb

        pending = start_chunk(0)
        for k in range(NCHUNK):
            p = k % 2
            nxt = start_chunk(k + 1) if k + 1 < NCHUNK else None
            pending[0].wait()
            pending[1].wait()

            @plsc.parallel_loop(0, CH, LANES, unroll=UNROLL)
            def _(off):
                w = w_v[pl.ds(k * CH + off, LANES)]
                ia = w & 0xFFFF
                ib = lax.shift_right_logical(w, 16)
                fa = fa_v[p, pl.ds(off, LANES)]
                fb = fb_v[p, pl.ds(off, LANES)]
                plsc.addupdate_scatter(acc_v, [ia], fa)
                plsc.addupdate_scatter(acc_v, [ib], fb)

            pending = nxt

        @plsc.parallel_loop(0, R3, LANES, unroll=UNROLL)
        def _(i):
            s = pl.ds(i, LANES)
            acc_v[s] = acc_v[s] * recip_v[s]

        pltpu.sync_copy(acc_v, out_hbm.at[b, ch])
        return carry

    lax.fori_loop(0, CPW, chan_body, 0)


_scatter = functools.partial(
    pl.kernel,
    mesh=plsc.VectorSubcoreMesh(core_axis_name="c", subcore_axis_name="s",
                                num_cores=2),
    compiler_params=pltpu.CompilerParams(needs_layout_passes=False),
    out_type=jax.ShapeDtypeStruct((B, C, R3), jnp.float32),
    scratch_types=[
        pltpu.VMEM((R3,), jnp.int32),
        pltpu.VMEM((R3,), jnp.float32),
        pltpu.VMEM((R3,), jnp.float32),
        pltpu.VMEM((2, CH), jnp.float32),
        pltpu.VMEM((2, CH), jnp.float32),
        pltpu.SemaphoreType.DMA,
        pltpu.SemaphoreType.DMA,
    ],
)(_scatter_body)


@jax.jit
def kernel(features, coords):
    norm_coords, w = _normalize(coords)
    vox = _scatter(features, w.reshape(B, HALF))
    return vox.reshape(B, C, R, R, R), norm_coords


# speedup vs baseline: 1.0583x; 1.0583x over previous
import functools

import jax
import jax.numpy as jnp
from jax import lax
from jax.experimental import pallas as pl
from jax.experimental.pallas import tpu as pltpu
from jax.experimental.pallas import tpu_sc as plsc

R = 32
R3 = R * R * R
B, C, N = 8, 32, 65536
HALF = N // 2
NW = 32
WPB = NW // B
CPW = C // WPB
CH = 4096
NCHUNK = HALF // CH
LANES = 16


def _normalize_body(coords_ref, norm_ref, w_ref):
    c = coords_ref[...]
    mean = jnp.mean(c, axis=2, keepdims=True)
    nc = c - mean
    r = jnp.max(jnp.sqrt(jnp.sum(nc * nc, axis=1, keepdims=True)),
                axis=2, keepdims=True)
    nc = nc / (r * 2.0) + 0.5
    nc = jnp.clip(nc * R, 0, R - 1)
    norm_ref[...] = nc
    vox = jnp.round(nc).astype(jnp.int32)
    idx = vox[:, 0, :] * (R * R) + vox[:, 1, :] * R + vox[:, 2, :]
    lo = idx[:, :HALF]
    hi = idx[:, HALF:]
    w_ref[...] = (lo | (hi << 16))[:, None, :]


_normalize = pl.pallas_call(
    _normalize_body,
    grid=(B,),
    in_specs=[pl.BlockSpec((1, 3, N), lambda b: (b, 0, 0))],
    out_specs=(
        pl.BlockSpec((1, 3, N), lambda b: (b, 0, 0)),
        pl.BlockSpec((1, 1, HALF), lambda b: (b, 0, 0)),
    ),
    out_shape=(
        jax.ShapeDtypeStruct((B, 3, N), jnp.float32),
        jax.ShapeDtypeStruct((B, 1, HALF), jnp.int32),
    ),
)


UNROLL = 8


def _scatter_body(feat_hbm, w_hbm, out_hbm, w_v, acc_v, recip_v,
                  fa0, fa1, fa2, fb0, fb1, fb2, sem_a, sem_b, sem_c):
    wid = lax.axis_index("s") * 2 + lax.axis_index("c")
    b = wid // WPB
    j = wid % WPB

    pltpu.sync_copy(w_hbm.at[b], w_v)

    zeros = jnp.zeros((LANES,), jnp.float32)
    ones = jnp.ones((LANES,), jnp.float32)

    def zero_acc():
        @plsc.parallel_loop(0, R3, LANES, unroll=UNROLL)
        def _(i):
            acc_v[pl.ds(i, LANES)] = zeros

    TOT = CPW * NCHUNK

    def start(g):
        c8, k = g // NCHUNK, g % NCHUNK
        p = g % 3
        ch = j * CPW + c8
        sem = (sem_a, sem_b, sem_c)[p]
        dst_a = (fa0, fa1, fa2)[p]
        dst_b = (fb0, fb1, fb2)[p]
        ca = pltpu.async_copy(feat_hbm.at[b, ch, pl.ds(k * CH, CH)],
                              dst_a, sem)
        cb = pltpu.async_copy(feat_hbm.at[b, ch, pl.ds(HALF + k * CH, CH)],
                              dst_b, sem)
        return ca, cb

    pend0 = start(0)
    pend1 = start(1)

    zero_acc()

    @plsc.parallel_loop(0, HALF, LANES, unroll=UNROLL)
    def _(i):
        w = w_v[pl.ds(i, LANES)]
        ia = w & 0xFFFF
        ib = lax.shift_right_logical(w, 16)
        plsc.addupdate_scatter(acc_v, [ia], ones)
        plsc.addupdate_scatter(acc_v, [ib], ones)

    @plsc.parallel_loop(0, R3, LANES, unroll=UNROLL)
    def _(i):
        s = pl.ds(i, LANES)
        recip_v[s] = 1.0 / jnp.maximum(acc_v[s], 1.0)

    pending = (pend0, pend1)
    for g in range(TOT):
        c8, k = g // NCHUNK, g % NCHUNK
        p = g % 3
        if k == 0:
            zero_acc()
        cur, ahead = pending
        nxt = start(g + 2) if g + 2 < TOT else None
        cur[0].wait()
        cur[1].wait()

        @plsc.parallel_loop(0, CH, LANES, unroll=UNROLL)
        def _(off):
            w = w_v[pl.ds(k * CH + off, LANES)]
            ia = w & 0xFFFF
            ib = lax.shift_right_logical(w, 16)
            fa = (fa0, fa1, fa2)[p][pl.ds(off, LANES)]
            fb = (fb0, fb1, fb2)[p][pl.ds(off, LANES)]
            plsc.addupdate_scatter(acc_v, [ia], fa)
            plsc.addupdate_scatter(acc_v, [ib], fb)

        pending = (ahead, nxt)

        if k == NCHUNK - 1:
            @plsc.parallel_loop(0, R3, LANES, unroll=UNROLL)
            def _(i):
                s = pl.ds(i, LANES)
                acc_v[s] = acc_v[s] * recip_v[s]

            pltpu.sync_copy(acc_v, out_hbm.at[b, j * CPW + c8])


_scatter = functools.partial(
    pl.kernel,
    mesh=plsc.VectorSubcoreMesh(core_axis_name="c", subcore_axis_name="s",
                                num_cores=2),
    compiler_params=pltpu.CompilerParams(needs_layout_passes=False),
    out_type=jax.ShapeDtypeStruct((B, C, R3), jnp.float32),
    scratch_types=[
        pltpu.VMEM((R3,), jnp.int32),
        pltpu.VMEM((R3,), jnp.float32),
        pltpu.VMEM((R3,), jnp.float32),
        pltpu.VMEM((CH,), jnp.float32),
        pltpu.VMEM((CH,), jnp.float32),
        pltpu.VMEM((CH,), jnp.float32),
        pltpu.VMEM((CH,), jnp.float32),
        pltpu.VMEM((CH,), jnp.float32),
        pltpu.VMEM((CH,), jnp.float32),
        pltpu.SemaphoreType.DMA,
        pltpu.SemaphoreType.DMA,
        pltpu.SemaphoreType.DMA,
    ],
)(_scatter_body)


@jax.jit
def kernel(features, coords):
    norm_coords, w = _normalize(coords)
    vox = _scatter(features, w.reshape(B, HALF))
    return vox.reshape(B, C, R, R, R), norm_coords
